# baseline (speedup 1.0000x reference)
import functools

import jax
import jax.numpy as jnp
from jax import lax
from jax.experimental import pallas as pl
from jax.experimental.pallas import tpu as pltpu
from jax.experimental.pallas import tpu_sc as plsc

_CH = 4096
_L = 16


def _gather_kernel(B, V, D, NC, NW):
    n_ch = B // _CH
    mesh = plsc.VectorSubcoreMesh(core_axis_name="c", subcore_axis_name="s")

    @functools.partial(
        pl.kernel,
        mesh=mesh,
        out_type=jax.ShapeDtypeStruct((D, B), jnp.float32),
        compiler_params=pltpu.CompilerParams(needs_layout_passes=False),
        scratch_types=[
            pltpu.VMEM((V,), jnp.float32),
            pltpu.VMEM((n_ch, _CH), jnp.int32),
            pltpu.VMEM((2, _CH), jnp.float32),
            [pltpu.SemaphoreType.DMA] * 2,
            [pltpu.SemaphoreType.DMA] * n_ch,
            [pltpu.SemaphoreType.DMA] * 2,
        ],
    )
    def k(idx_hbm, tab_hbm, out_hbm, col_v, idx_v, out_v, csems, isems, osems):
        c = lax.axis_index("s") * NC + lax.axis_index("c")
        idx_cps = [
            pltpu.async_copy(
                idx_hbm.at[pl.ds(ch * _CH, _CH)], idx_v.at[ch], isems[ch]
            )
            for ch in range(n_ch)
        ]
        pltpu.async_copy(tab_hbm.at[c], col_v, csems[0]).wait()
        wb = [None, None]
        for ch in range(n_ch):
            idx_cps[ch].wait()
            if wb[ch % 2] is not None:
                wb[ch % 2].wait()

            @plsc.parallel_loop(0, _CH, step=_L, unroll=16)
            def _(t):
                out_v[ch % 2, pl.ds(t, _L)] = plsc.load_gather(
                    col_v, [idx_v[ch, pl.ds(t, _L)]]
                )

            wb[ch % 2] = pltpu.async_copy(
                out_v.at[ch % 2],
                out_hbm.at[c, pl.ds(ch * _CH, _CH)],
                osems[ch % 2],
            )
        for cp in wb:
            cp.wait()

    return k


def kernel(domains, table):
    B, = domains.shape
    V, D = table.shape
    info = plsc.get_sparse_core_info()
    NC, NS = info.num_cores, info.num_subcores
    NW = NC * NS
    k = _gather_kernel(B, V, D, NC, NW)
    return k(domains, table.T).T


# speedup vs baseline: 1.0168x; 1.0168x over previous
import functools

import jax
import jax.numpy as jnp
from jax import lax
from jax.experimental import pallas as pl
from jax.experimental.pallas import tpu as pltpu
from jax.experimental.pallas import tpu_sc as plsc

_CH = 4096
_L = 16


def _gather_kernel(B, V, D, NC, NW):
    n_ch = B // _CH
    mesh = plsc.VectorSubcoreMesh(core_axis_name="c", subcore_axis_name="s")

    @functools.partial(
        pl.kernel,
        mesh=mesh,
        out_type=jax.ShapeDtypeStruct((D, B), jnp.float32),
        compiler_params=pltpu.CompilerParams(needs_layout_passes=False),
        scratch_types=[
            pltpu.VMEM((V,), jnp.float32),
            pltpu.VMEM((n_ch, _CH), jnp.int32),
            pltpu.VMEM((2, _CH), jnp.float32),
            [pltpu.SemaphoreType.DMA] * 2,
            [pltpu.SemaphoreType.DMA] * n_ch,
            [pltpu.SemaphoreType.DMA] * 2,
        ],
    )
    def k(idx_hbm, tab_hbm, out_hbm, col_v, idx_v, out_v, csems, isems, osems):
        c = lax.axis_index("s") * NC + lax.axis_index("c")
        col_cp = pltpu.async_copy(tab_hbm.at[c], col_v, csems[0])
        idx_cps = [
            pltpu.async_copy(
                idx_hbm.at[pl.ds(ch * _CH, _CH)], idx_v.at[ch], isems[ch]
            )
            for ch in range(n_ch)
        ]
        col_cp.wait()
        wb = [None, None]
        for ch in range(n_ch):
            idx_cps[ch].wait()
            if wb[ch % 2] is not None:
                wb[ch % 2].wait()

            @plsc.parallel_loop(0, _CH, step=_L, unroll=8)
            def _(t):
                out_v[ch % 2, pl.ds(t, _L)] = plsc.load_gather(
                    col_v, [idx_v[ch, pl.ds(t, _L)]]
                )

            wb[ch % 2] = pltpu.async_copy(
                out_v.at[ch % 2],
                out_hbm.at[c, pl.ds(ch * _CH, _CH)],
                osems[ch % 2],
            )
        for cp in wb:
            cp.wait()

    return k


def kernel(domains, table):
    B, = domains.shape
    V, D = table.shape
    info = plsc.get_sparse_core_info()
    NC, NS = info.num_cores, info.num_subcores
    NW = NC * NS
    k = _gather_kernel(B, V, D, NC, NW)
    return k(domains, table.T).T


# speedup vs baseline: 1.0224x; 1.0056x over previous
import functools

import jax
import jax.numpy as jnp
from jax import lax
from jax.experimental import pallas as pl
from jax.experimental.pallas import tpu as pltpu
from jax.experimental.pallas import tpu_sc as plsc

_CH = 4096
_L = 16


def _gather_kernel(B, V, D, NC, NW):
    n_ch = B // _CH
    mesh = plsc.VectorSubcoreMesh(core_axis_name="c", subcore_axis_name="s")

    @functools.partial(
        pl.kernel,
        mesh=mesh,
        out_type=jax.ShapeDtypeStruct((D, B), jnp.float32),
        compiler_params=pltpu.CompilerParams(needs_layout_passes=False),
        scratch_types=[
            pltpu.VMEM((V,), jnp.float32),
            pltpu.VMEM((B,), jnp.int32),
            pltpu.VMEM((2, _CH), jnp.float32),
            [pltpu.SemaphoreType.DMA] * 2,
            [pltpu.SemaphoreType.DMA] * 2,
        ],
    )
    def k(idx_hbm, tab_hbm, out_hbm, col_v, idx_v, out_v, csems, osems):
        c = lax.axis_index("s") * NC + lax.axis_index("c")
        col_cp = pltpu.async_copy(tab_hbm.at[c], col_v, csems[0])
        idx_cp = pltpu.async_copy(idx_hbm, idx_v, csems[1])
        idx_cp.wait()
        col_cp.wait()
        wb = [None, None]
        for ch in range(n_ch):
            if wb[ch % 2] is not None:
                wb[ch % 2].wait()

            @plsc.parallel_loop(0, _CH, step=_L, unroll=8)
            def _(t):
                out_v[ch % 2, pl.ds(t, _L)] = plsc.load_gather(
                    col_v, [idx_v[pl.ds(ch * _CH + t, _L)]]
                )

            wb[ch % 2] = pltpu.async_copy(
                out_v.at[ch % 2],
                out_hbm.at[c, pl.ds(ch * _CH, _CH)],
                osems[ch % 2],
            )
        for cp in wb:
            cp.wait()

    return k


def kernel(domains, table):
    B, = domains.shape
    V, D = table.shape
    info = plsc.get_sparse_core_info()
    NC, NS = info.num_cores, info.num_subcores
    NW = NC * NS
    k = _gather_kernel(B, V, D, NC, NW)
    return k(domains, table.T).T


# speedup vs baseline: 1.1242x; 1.0996x over previous
import functools

import jax
import jax.numpy as jnp
from jax import lax
from jax.experimental import pallas as pl
from jax.experimental.pallas import tpu as pltpu
from jax.experimental.pallas import tpu_sc as plsc

_CH = 4096
_L = 16


def _gather_kernel(B, V, D, NC, NW):
    n_ch = B // _CH
    mesh = plsc.VectorSubcoreMesh(core_axis_name="c", subcore_axis_name="s")

    @functools.partial(
        pl.kernel,
        mesh=mesh,
        out_type=jax.ShapeDtypeStruct((D, B), jnp.float32),
        compiler_params=pltpu.CompilerParams(needs_layout_passes=False),
        scratch_types=[
            pltpu.VMEM((V,), jnp.float32),
            pltpu.VMEM((B,), jnp.int32),
            pltpu.VMEM((2, _CH), jnp.float32),
            pltpu.VMEM_SHARED((B,), jnp.int32),
            [pltpu.SemaphoreType.DMA] * 2,
            [pltpu.SemaphoreType.DMA] * 2,
        ],
    )
    def k(idx_hbm, tab_hbm, out_hbm, col_v, idx_v, out_v, idx_sh, csems, osems):
        c = lax.axis_index("s") * NC + lax.axis_index("c")
        s = lax.axis_index("s")
        col_cp = pltpu.async_copy(tab_hbm.at[c], col_v, csems[0])
        @pl.when(s == 0)
        def _():
            pltpu.sync_copy(idx_hbm, idx_sh)

        plsc.subcore_barrier()
        pltpu.sync_copy(idx_sh, idx_v)
        col_cp.wait()
        wb = [None, None]
        for ch in range(n_ch):
            if wb[ch % 2] is not None:
                wb[ch % 2].wait()

            @plsc.parallel_loop(0, _CH, step=_L, unroll=8)
            def _(t):
                out_v[ch % 2, pl.ds(t, _L)] = plsc.load_gather(
                    col_v, [idx_v[pl.ds(ch * _CH + t, _L)]]
                )

            wb[ch % 2] = pltpu.async_copy(
                out_v.at[ch % 2],
                out_hbm.at[c, pl.ds(ch * _CH, _CH)],
                osems[ch % 2],
            )
        for cp in wb:
            cp.wait()

    return k


def kernel(domains, table):
    B, = domains.shape
    V, D = table.shape
    info = plsc.get_sparse_core_info()
    NC, NS = info.num_cores, info.num_subcores
    NW = NC * NS
    k = _gather_kernel(B, V, D, NC, NW)
    return k(domains, table.T).T
